# baseline (speedup 1.0000x reference)
import jax
import jax.numpy as jnp
from jax.experimental import pallas as pl
from jax.experimental.pallas import tpu as pltpu
from jax.experimental.pallas import tpu_sc as plsc

_MAX_SEQLEN = 4096
_SEG_W = 4096
_TQ = 512
_TK = 2048
_NSHIFT = 8


def _tc_add_kernel(seg_ref, attn_ref, out_ref, u_ref):
    i = pl.program_id(1)

    @pl.when(i == 0)
    def _build_u():
        seg = seg_ref[0, 0, :]
        v = jnp.broadcast_to(seg[None, :], (8, _SEG_W))
        row = jax.lax.broadcasted_iota(jnp.int32, (8, _SEG_W), 0)
        for t in range(3):
            n = 1 << t
            rolled = pltpu.roll(v, _SEG_W - n, axis=1)
            v = jnp.where((row & n) == 0, rolled, v)
        for a in range(_TQ // 8):
            n = _TQ - 8 - 8 * a
            slab = pltpu.roll(v, _SEG_W - n, axis=1) if n else v
            u_ref[8 * a:8 * (a + 1), :] = slab

    for ii in range(2048 // _TQ):
        @pl.when(i == ii)
        def _consume(ii=ii):
            x0 = (_MAX_SEQLEN // 2 - _TQ) - _TQ * ii
            out_ref[0, 0] = attn_ref[0, 0] + u_ref[:, x0:x0 + _TK]


def _tc_out(attn_mtx, seg):
    b, h, q, k = attn_mtx.shape
    blk = pl.BlockSpec((1, 1, _TQ, _TK), lambda hh, ii: (0, hh, ii, 0))
    return pl.pallas_call(
        _tc_add_kernel,
        grid=(h, q // _TQ),
        in_specs=[
            pl.BlockSpec((1, 1, _SEG_W), lambda hh, ii: (hh, 0, 0)),
            blk,
        ],
        out_specs=blk,
        out_shape=jax.ShapeDtypeStruct((b, h, q, k), jnp.float32),
        scratch_shapes=[pltpu.VMEM((_TQ, _SEG_W), jnp.float32)],
        compiler_params=pltpu.CompilerParams(
            dimension_semantics=("parallel", "arbitrary"),
        ),
    )(seg, attn_mtx)


def _sc_bias(seg_shift_flat, nh, nq, nk):
    mesh = plsc.VectorSubcoreMesh(core_axis_name="c", subcore_axis_name="s")
    rows_per_unit = nh * nq // 32
    ngroups = rows_per_unit // _NSHIFT

    @pl.kernel(
        out_type=jax.ShapeDtypeStruct((nh * nq * nk,), jnp.float32),
        mesh=mesh,
        scratch_types=(
            [pltpu.VMEM((_SEG_W,), jnp.float32) for _ in range(_NSHIFT)]
            + [pltpu.SemaphoreType.DMA]
        ),
    )
    def body(seg_hbm, bias_hbm, *scr):
        segs, sem = scr[:_NSHIFT], scr[_NSHIFT]
        c = jax.lax.axis_index("c")
        s = jax.lax.axis_index("s")
        u = c * 16 + s
        h = u // 2
        q0 = (u % 2) * rows_per_unit
        for j in range(_NSHIFT):
            pltpu.sync_copy(
                seg_hbm.at[pl.ds((j * nh + h) * _SEG_W, _SEG_W)], segs[j]
            )

        row0 = (h * nq + q0) * nk

        def group(g, wait):
            base = pl.multiple_of((nq - _NSHIFT) - q0 - _NSHIFT * g, _NSHIFT)
            dst0 = pl.multiple_of(row0 + _NSHIFT * g * nk, _NSHIFT)
            for e in range(_NSHIFT):
                pltpu.make_async_copy(
                    segs[_NSHIFT - 1 - e].at[pl.ds(base, nk)],
                    bias_hbm.at[pl.ds(dst0 + e * nk, nk)],
                    sem,
                ).start()
            if wait:
                for _ in range(_NSHIFT):
                    pltpu.make_async_copy(
                        segs[0].at[pl.ds(0, nk)],
                        bias_hbm.at[pl.ds(row0, nk)],
                        sem,
                    ).wait()

        group(0, False)

        @pl.loop(1, ngroups)
        def _steady(g):
            group(g, True)

        for _ in range(_NSHIFT):
            pltpu.make_async_copy(
                segs[0].at[pl.ds(0, nk)],
                bias_hbm.at[pl.ds(row0, nk)],
                sem,
            ).wait()

    return body(seg_shift_flat)


def kernel(attn_mtx, embedding_table):
    b, h, q, k = attn_mtx.shape
    assert (b, h, q, k) == (1, 16, 2048, 2048)
    seg2d = jnp.flip(embedding_table[2048:6143, :], axis=0).T
    segp = jnp.pad(seg2d, ((0, 0), (0, _SEG_W + _NSHIFT - seg2d.shape[1])))
    seg_shift = jnp.stack(
        [segp[:, j:j + _SEG_W] for j in range(_NSHIFT)], axis=0
    )
    seg3d = segp[:, None, :_SEG_W]

    bias = _sc_bias(seg_shift.reshape(-1), h, q, k).reshape(b, h, q, k)
    out = _tc_out(attn_mtx, seg3d)
    return out, bias
